# baseline (speedup 1.0000x reference)
import functools

import jax
import jax.numpy as jnp
from jax import lax
from jax.experimental import pallas as pl
from jax.experimental.pallas import tpu as pltpu
from jax.experimental.pallas import tpu_sc as plsc

N = 10000
E = 320000
NP = 10240
CH = 64
SUP = 9
NCH = 162
NTILES = 32
EP = NTILES * CH * NCH
ROW_BLK = 2000
ACCW = 136


def _permute16(x, idx):
    dnums = lax.GatherDimensionNumbers(
        offset_dims=(), collapsed_slice_dims=(0,), start_index_map=(0,))
    return lax.gather(x, idx[:, None], dnums, (1,),
                      mode=lax.GatherScatterMode.PROMISE_IN_BOUNDS)


def _make_edge_kernel(heads):
    mesh = plsc.VectorSubcoreMesh(core_axis_name="c", subcore_axis_name="s")
    rows_per_sub = NP // 16

    buf_scratch = [
        pltpu.VMEM((CH, 16), jnp.float32),
        pltpu.VMEM((CH, 144), jnp.float32),
        pltpu.VMEM((CH, ACCW), jnp.float32),
        pltpu.VMEM((CH,), jnp.int32),
    ]
    sup_scratch = [
        pltpu.VMEM((SUP * CH,), jnp.int32),
        pltpu.VMEM((SUP * CH,), jnp.int32),
    ]

    @functools.partial(
        pl.kernel,
        mesh=mesh,
        compiler_params=pltpu.CompilerParams(use_tc_tiling_on_sc=False),
        out_type=jax.ShapeDtypeStruct((2, NP, ACCW), jnp.float32),
        scratch_types=buf_scratch + buf_scratch + sup_scratch + sup_scratch + [
            pltpu.VMEM((8, ACCW), jnp.float32),
            pltpu.VMEM_SHARED((NP, ACCW), jnp.float32),
            pltpu.SemaphoreType.DMA,
            pltpu.SemaphoreType.DMA,
            pltpu.SemaphoreType.DMA,
            pltpu.SemaphoreType.DMA,
            pltpu.SemaphoreType.DMA,
            pltpu.SemaphoreType.DMA,
        ],
    )
    def edge_kernel(a_hbm, h_hbm, src_hbm, dst_hbm, out_hbm,
                    v0, hx0, m0, w0, v1, hx1, m1, w1,
                    ss0, sd0, ss1, sd1,
                    z_v, acc_sh, g0, g1, q0, q1, i0, i1):
        bufs = ((v0, hx0, m0, w0, g0, q0),
                (v1, hx1, m1, w1, g1, q1))
        sups = ((ss0, sd0, i0), (ss1, sd1, i1))
        cid = lax.axis_index("c")
        sid = lax.axis_index("s")
        wid = cid * 16 + sid
        lane = lax.iota(jnp.int32, 16)
        zero16 = jnp.zeros((16,), jnp.float32)

        for r in range(8):
            for cc in range(8):
                z_v[r, pl.ds(cc * 16, 16)] = zero16
            z_v[r, pl.ds(ACCW - 16, 16)] = zero16
        row0 = sid * rows_per_sub
        for kk in range(rows_per_sub // 8):
            pltpu.sync_copy(z_v, acc_sh.at[pl.ds(row0 + kk * 8, 8)])
        plsc.subcore_barrier()

        rot = 8 + (lane & 7)
        pmask = lane < heads
        tile_base = wid * (NCH * CH)

        def issue_sup(s, sp):
            base = tile_base + s * (SUP * CH)
            pltpu.async_copy(src_hbm.at[pl.ds(base, SUP * CH)], sp[0], sp[2])
            pltpu.async_copy(dst_hbm.at[pl.ds(base, SUP * CH)], sp[1], sp[2])

        def wait_sup(s, sp):
            base = tile_base + s * (SUP * CH)
            pltpu.make_async_copy(
                src_hbm.at[pl.ds(base, SUP * CH)], sp[0], sp[2]).wait()
            pltpu.make_async_copy(
                dst_hbm.at[pl.ds(base, SUP * CH)], sp[1], sp[2]).wait()

        def issue_gathers(coff, sp, S):
            v_v, hx_v = S[0], S[1]
            pltpu.async_copy(a_hbm.at[sp[1].at[pl.ds(coff, CH)]], v_v, S[4])
            pltpu.async_copy(h_hbm.at[sp[0].at[pl.ds(coff, CH)]], hx_v, S[4])

        def wait_gathers(coff, sp, S):
            v_v, hx_v = S[0], S[1]
            pltpu.make_async_copy(
                a_hbm.at[sp[1].at[pl.ds(coff, CH)]], v_v, S[4]).wait()
            pltpu.make_async_copy(
                h_hbm.at[sp[0].at[pl.ds(coff, CH)]], hx_v, S[4]).wait()

        def wait_scatter(S):
            pltpu.make_async_copy(S[2], acc_sh.at[S[3]], S[5]).wait()

        def compute_scatter(coff, sp, S):
            v_v, hx_v, m_v, w_v = S[0], S[1], S[2], S[3]
            for k in range(CH // 16):
                w_v[pl.ds(k * 16, 16)] = sp[1][pl.ds(coff + k * 16, 16)]

            @plsc.parallel_loop(0, CH, unroll=4)
            def edge_body(j):
                u = hx_v[j, pl.ds(128, 16)]
                v = v_v[j]
                if heads == 8:
                    e = u + _permute16(v, rot)
                else:
                    e = (_permute16(u, lane * 0)
                         + _permute16(v, lane * 0 + 1))
                e = jnp.maximum(e, 0.2 * e)
                p = jnp.exp(e)
                m_v[j, pl.ds(0, 16)] = jnp.where(pmask, p, 0.0)
                for jh in range(8):
                    if heads == 8:
                        pj = _permute16(p, lane * 0 + jh)
                    else:
                        pj = p
                    m_v[j, pl.ds(8 + jh * 16, 16)] = (
                        hx_v[j, pl.ds(jh * 16, 16)] * pj)

            pltpu.async_copy(m_v, acc_sh.at[w_v], S[5], add=True)

        def pair_body(t, carry):
            for k in range(18):
                kc = k % 9
                sp_cur = sups[(k // 9) % 2]
                S = bufs[k % 2]
                Y = bufs[(k + 1) % 2]
                wait_gathers(kc * CH, sp_cur, S)
                if k == 0:
                    issue_sup(2 * t + 1, sups[1])
                if k == 9:
                    @pl.when(t < NCH // 18 - 1)
                    def _():
                        issue_sup(2 * t + 2, sups[0])
                if k in (0, 1):
                    @pl.when(t > 0)
                    def _():
                        wait_scatter(S)
                else:
                    wait_scatter(S)
                if k == 17:
                    @pl.when(t < NCH // 18 - 1)
                    def _():
                        wait_sup(2 * t + 2, sups[0])
                        issue_gathers(0, sups[0], Y)
                else:
                    if k == 8:
                        wait_sup(2 * t + 1, sups[1])
                        issue_gathers(0, sups[1], Y)
                    else:
                        issue_gathers((kc + 1) * CH,
                                      sups[(k // 9) % 2], Y)
                compute_scatter(kc * CH, sp_cur, S)
            return carry

        issue_sup(0, sups[0])
        wait_sup(0, sups[0])
        issue_gathers(0, sups[0], bufs[0])
        lax.fori_loop(0, NCH // 18, pair_body, 0)
        wait_scatter(bufs[0])
        wait_scatter(bufs[1])

        plsc.subcore_barrier()
        pltpu.sync_copy(acc_sh.at[pl.ds(row0, rows_per_sub)],
                        out_hbm.at[cid, pl.ds(row0, rows_per_sub)])

    return edge_kernel


_edge_sc_8 = _make_edge_kernel(8)
_edge_sc_1 = _make_edge_kernel(1)


def _mm_attn_body(x_ref, w_ref, a_ref, h_ref, out_a_ref):
    h = x_ref[...] @ w_ref[...]
    a = h @ a_ref[...]
    h_ref[:, :128] = h
    h_ref[:, 128:144] = a
    out_a_ref[...] = a


def _mm_attn(x, W, A):
    n, f = x.shape
    k = A.shape[1]
    return pl.pallas_call(
        _mm_attn_body,
        grid=(n // ROW_BLK,),
        in_specs=[
            pl.BlockSpec((ROW_BLK, f), lambda i: (i, 0)),
            pl.BlockSpec((f, 128), lambda i: (0, 0)),
            pl.BlockSpec((128, k), lambda i: (0, 0)),
        ],
        out_specs=[
            pl.BlockSpec((ROW_BLK, 144), lambda i: (i, 0)),
            pl.BlockSpec((ROW_BLK, k), lambda i: (i, 0)),
        ],
        out_shape=[
            jax.ShapeDtypeStruct((NP, 144), jnp.float32),
            jax.ShapeDtypeStruct((NP, k), jnp.float32),
        ],
    )(x, W, A)


def _norm_elu_mm_body(s_ref, r_ref, b_ref, w_ref, a_ref, g_ref, a2_ref):
    acc = s_ref[0] + s_ref[1]
    msg = acc[:, 8:136]
    den = acc[:, 0:8]
    dex = den @ r_ref[...]
    out1 = msg / (dex + 1e-16)
    xb = out1 + b_ref[...]
    h1 = jnp.where(xb > 0, xb, jnp.exp(jnp.minimum(xb, 0.0)) - 1.0)
    g = h1 @ w_ref[...]
    a2 = g @ a_ref[...]
    g_ref[:, :128] = g
    g_ref[:, 128:144] = a2
    a2_ref[...] = a2


def _norm_elu_mm(s, R816, b, W, A):
    k = A.shape[1]
    return pl.pallas_call(
        _norm_elu_mm_body,
        grid=(N // ROW_BLK,),
        in_specs=[
            pl.BlockSpec((2, ROW_BLK, ACCW), lambda i: (0, i, 0)),
            pl.BlockSpec((8, 128), lambda i: (0, 0)),
            pl.BlockSpec((1, 128), lambda i: (0, 0)),
            pl.BlockSpec((128, 128), lambda i: (0, 0)),
            pl.BlockSpec((128, k), lambda i: (0, 0)),
        ],
        out_specs=[
            pl.BlockSpec((ROW_BLK, 144), lambda i: (i, 0)),
            pl.BlockSpec((ROW_BLK, k), lambda i: (i, 0)),
        ],
        out_shape=[
            jax.ShapeDtypeStruct((NP, 144), jnp.float32),
            jax.ShapeDtypeStruct((NP, k), jnp.float32),
        ],
    )(s, R816, b.reshape(1, 128), W, A)


def _final_body(s_ref, b_ref, o_ref):
    acc = s_ref[0] + s_ref[1]
    msg = acc[:, 8:136]
    den = acc[:, 0:1]
    o_ref[...] = msg / (den + 1e-16) + b_ref[...]


def _final(s, b):
    return pl.pallas_call(
        _final_body,
        grid=(N // ROW_BLK,),
        in_specs=[
            pl.BlockSpec((2, ROW_BLK, ACCW), lambda i: (0, i, 0)),
            pl.BlockSpec((1, 128), lambda i: (0, 0)),
        ],
        out_specs=pl.BlockSpec((ROW_BLK, 128), lambda i: (i, 0)),
        out_shape=jax.ShapeDtypeStruct((N, 128), jnp.float32),
    )(s, b.reshape(1, 128))


def kernel(x, edge_index, W1, att_src1, att_dst1, b1, W2, att_src2, att_dst2, b2):
    n = x.shape[0]
    loop = jnp.arange(n, dtype=edge_index.dtype)
    pad = EP - (E + n)
    src_all = jnp.concatenate(
        [edge_index[0], loop, jnp.full((pad,), n, edge_index.dtype)])
    dst_all = jnp.concatenate(
        [edge_index[1], loop, jnp.full((pad,), n, edge_index.dtype)])

    H1, C1 = att_src1.shape
    eye1 = jnp.eye(H1, dtype=x.dtype)
    As1 = (eye1[:, None, :] * att_src1[:, :, None]).reshape(H1 * C1, H1)
    Ad1 = (eye1[:, None, :] * att_dst1[:, :, None]).reshape(H1 * C1, H1)
    A1 = jnp.concatenate([As1, Ad1], axis=1)
    A2 = jnp.concatenate(
        [att_src2.T, att_dst2.T,
         jnp.zeros((128, 14), x.dtype)], axis=1)
    R816 = jnp.repeat(jnp.eye(8, dtype=x.dtype), 16, axis=1)

    h, a1 = _mm_attn(x, W1, A1)
    s1 = _edge_sc_8(a1, h, src_all, dst_all)
    g, a2 = _norm_elu_mm(s1, R816, b1, W2, A2)
    s2 = _edge_sc_1(a2, g, src_all, dst_all)
    return _final(s2, b2)


# speedup vs baseline: 1.0174x; 1.0174x over previous
import functools

import jax
import jax.numpy as jnp
from jax import lax
from jax.experimental import pallas as pl
from jax.experimental.pallas import tpu as pltpu
from jax.experimental.pallas import tpu_sc as plsc

N = 10000
E = 320000
NP = 10240
CH = 64
SUP = 9
NCH = 162
NTILES = 32
EP = NTILES * CH * NCH
ROW_BLK = 2000
ACCW = 136


def _permute16(x, idx):
    dnums = lax.GatherDimensionNumbers(
        offset_dims=(), collapsed_slice_dims=(0,), start_index_map=(0,))
    return lax.gather(x, idx[:, None], dnums, (1,),
                      mode=lax.GatherScatterMode.PROMISE_IN_BOUNDS)


def _make_edge_kernel(heads):
    mesh = plsc.VectorSubcoreMesh(core_axis_name="c", subcore_axis_name="s")
    rows_per_sub = NP // 16

    buf_scratch = [
        pltpu.VMEM((CH, 16), jnp.float32),
        pltpu.VMEM((CH, 144), jnp.float32),
        pltpu.VMEM((CH, ACCW), jnp.float32),
        pltpu.VMEM((CH,), jnp.int32),
    ]
    sup_scratch = [
        pltpu.VMEM((SUP * CH,), jnp.int32),
        pltpu.VMEM((SUP * CH,), jnp.int32),
    ]

    @functools.partial(
        pl.kernel,
        mesh=mesh,
        compiler_params=pltpu.CompilerParams(use_tc_tiling_on_sc=False),
        out_type=jax.ShapeDtypeStruct((2, NP, ACCW), jnp.float32),
        scratch_types=buf_scratch + buf_scratch + sup_scratch + sup_scratch + [
            pltpu.VMEM((8, ACCW), jnp.float32),
            pltpu.VMEM_SHARED((NP, ACCW), jnp.float32),
            pltpu.SemaphoreType.DMA,
            pltpu.SemaphoreType.DMA,
            pltpu.SemaphoreType.DMA,
            pltpu.SemaphoreType.DMA,
            pltpu.SemaphoreType.DMA,
            pltpu.SemaphoreType.DMA,
        ],
    )
    def edge_kernel(a_hbm, h_hbm, src_hbm, dst_hbm, out_hbm,
                    v0, hx0, m0, w0, v1, hx1, m1, w1,
                    ss0, sd0, ss1, sd1,
                    z_v, acc_sh, g0, g1, q0, q1, i0, i1):
        bufs = ((v0, hx0, m0, w0, g0, q0),
                (v1, hx1, m1, w1, g1, q1))
        sups = ((ss0, sd0, i0), (ss1, sd1, i1))
        cid = lax.axis_index("c")
        sid = lax.axis_index("s")
        wid = cid * 16 + sid
        lane = lax.iota(jnp.int32, 16)
        zero16 = jnp.zeros((16,), jnp.float32)

        for r in range(8):
            for cc in range(8):
                z_v[r, pl.ds(cc * 16, 16)] = zero16
            z_v[r, pl.ds(ACCW - 16, 16)] = zero16
        row0 = sid * rows_per_sub
        nz = rows_per_sub // 8
        for kk0 in range(0, nz, 16):
            kb = range(kk0, min(kk0 + 16, nz))
            for kk in kb:
                pltpu.async_copy(z_v, acc_sh.at[pl.ds(row0 + kk * 8, 8)], i0)
            for kk in kb:
                pltpu.make_async_copy(
                    z_v, acc_sh.at[pl.ds(row0 + kk * 8, 8)], i0).wait()
        plsc.subcore_barrier()

        rot = 8 + (lane & 7)
        pmask = lane < heads
        tile_base = wid * (NCH * CH)

        def issue_sup(s, sp):
            base = tile_base + s * (SUP * CH)
            pltpu.async_copy(src_hbm.at[pl.ds(base, SUP * CH)], sp[0], sp[2])
            pltpu.async_copy(dst_hbm.at[pl.ds(base, SUP * CH)], sp[1], sp[2])

        def wait_sup(s, sp):
            base = tile_base + s * (SUP * CH)
            pltpu.make_async_copy(
                src_hbm.at[pl.ds(base, SUP * CH)], sp[0], sp[2]).wait()
            pltpu.make_async_copy(
                dst_hbm.at[pl.ds(base, SUP * CH)], sp[1], sp[2]).wait()

        def issue_gathers(coff, sp, S):
            v_v, hx_v = S[0], S[1]
            pltpu.async_copy(a_hbm.at[sp[1].at[pl.ds(coff, CH)]], v_v, S[4])
            pltpu.async_copy(h_hbm.at[sp[0].at[pl.ds(coff, CH)]], hx_v, S[4])

        def wait_gathers(coff, sp, S):
            v_v, hx_v = S[0], S[1]
            pltpu.make_async_copy(
                a_hbm.at[sp[1].at[pl.ds(coff, CH)]], v_v, S[4]).wait()
            pltpu.make_async_copy(
                h_hbm.at[sp[0].at[pl.ds(coff, CH)]], hx_v, S[4]).wait()

        def wait_scatter(S):
            pltpu.make_async_copy(S[2], acc_sh.at[S[3]], S[5]).wait()

        def compute_scatter(coff, sp, S):
            v_v, hx_v, m_v, w_v = S[0], S[1], S[2], S[3]
            for k in range(CH // 16):
                w_v[pl.ds(k * 16, 16)] = sp[1][pl.ds(coff + k * 16, 16)]

            @plsc.parallel_loop(0, CH, unroll=2)
            def edge_body(j):
                u = hx_v[j, pl.ds(128, 16)]
                v = v_v[j]
                if heads == 8:
                    e = u + _permute16(v, rot)
                else:
                    e = (_permute16(u, lane * 0)
                         + _permute16(v, lane * 0 + 1))
                e = jnp.maximum(e, 0.2 * e)
                p = jnp.exp(e)
                m_v[j, pl.ds(0, 16)] = jnp.where(pmask, p, 0.0)
                for jh in range(8):
                    if heads == 8:
                        pj = _permute16(p, lane * 0 + jh)
                    else:
                        pj = p
                    m_v[j, pl.ds(8 + jh * 16, 16)] = (
                        hx_v[j, pl.ds(jh * 16, 16)] * pj)

            pltpu.async_copy(m_v, acc_sh.at[w_v], S[5], add=True)

        def pair_body(t, carry):
            for k in range(18):
                kc = k % 9
                sp_cur = sups[(k // 9) % 2]
                S = bufs[k % 2]
                Y = bufs[(k + 1) % 2]
                wait_gathers(kc * CH, sp_cur, S)
                if k == 0:
                    issue_sup(2 * t + 1, sups[1])
                if k == 9:
                    @pl.when(t < NCH // 18 - 1)
                    def _():
                        issue_sup(2 * t + 2, sups[0])
                if k in (0, 1):
                    @pl.when(t > 0)
                    def _():
                        wait_scatter(S)
                else:
                    wait_scatter(S)
                if k == 17:
                    @pl.when(t < NCH // 18 - 1)
                    def _():
                        wait_sup(2 * t + 2, sups[0])
                        issue_gathers(0, sups[0], Y)
                else:
                    if k == 8:
                        wait_sup(2 * t + 1, sups[1])
                        issue_gathers(0, sups[1], Y)
                    else:
                        issue_gathers((kc + 1) * CH,
                                      sups[(k // 9) % 2], Y)
                compute_scatter(kc * CH, sp_cur, S)
            return carry

        issue_sup(0, sups[0])
        wait_sup(0, sups[0])
        issue_gathers(0, sups[0], bufs[0])
        lax.fori_loop(0, NCH // 18, pair_body, 0)
        wait_scatter(bufs[0])
        wait_scatter(bufs[1])

        plsc.subcore_barrier()
        pltpu.sync_copy(acc_sh.at[pl.ds(row0, rows_per_sub)],
                        out_hbm.at[cid, pl.ds(row0, rows_per_sub)])

    return edge_kernel


_edge_sc_8 = _make_edge_kernel(8)
_edge_sc_1 = _make_edge_kernel(1)


def _mm_attn_body(x_ref, w_ref, a_ref, h_ref, out_a_ref):
    h = x_ref[...] @ w_ref[...]
    a = h @ a_ref[...]
    h_ref[:, :128] = h
    h_ref[:, 128:144] = a
    out_a_ref[...] = a


def _mm_attn(x, W, A):
    n, f = x.shape
    k = A.shape[1]
    return pl.pallas_call(
        _mm_attn_body,
        grid=(n // ROW_BLK,),
        in_specs=[
            pl.BlockSpec((ROW_BLK, f), lambda i: (i, 0)),
            pl.BlockSpec((f, 128), lambda i: (0, 0)),
            pl.BlockSpec((128, k), lambda i: (0, 0)),
        ],
        out_specs=[
            pl.BlockSpec((ROW_BLK, 144), lambda i: (i, 0)),
            pl.BlockSpec((ROW_BLK, k), lambda i: (i, 0)),
        ],
        out_shape=[
            jax.ShapeDtypeStruct((NP, 144), jnp.float32),
            jax.ShapeDtypeStruct((NP, k), jnp.float32),
        ],
    )(x, W, A)


def _norm_elu_mm_body(s_ref, r_ref, b_ref, w_ref, a_ref, g_ref, a2_ref):
    acc = s_ref[0] + s_ref[1]
    msg = acc[:, 8:136]
    den = acc[:, 0:8]
    dex = den @ r_ref[...]
    out1 = msg / (dex + 1e-16)
    xb = out1 + b_ref[...]
    h1 = jnp.where(xb > 0, xb, jnp.exp(jnp.minimum(xb, 0.0)) - 1.0)
    g = h1 @ w_ref[...]
    a2 = g @ a_ref[...]
    g_ref[:, :128] = g
    g_ref[:, 128:144] = a2
    a2_ref[...] = a2


def _norm_elu_mm(s, R816, b, W, A):
    k = A.shape[1]
    return pl.pallas_call(
        _norm_elu_mm_body,
        grid=(N // ROW_BLK,),
        in_specs=[
            pl.BlockSpec((2, ROW_BLK, ACCW), lambda i: (0, i, 0)),
            pl.BlockSpec((8, 128), lambda i: (0, 0)),
            pl.BlockSpec((1, 128), lambda i: (0, 0)),
            pl.BlockSpec((128, 128), lambda i: (0, 0)),
            pl.BlockSpec((128, k), lambda i: (0, 0)),
        ],
        out_specs=[
            pl.BlockSpec((ROW_BLK, 144), lambda i: (i, 0)),
            pl.BlockSpec((ROW_BLK, k), lambda i: (i, 0)),
        ],
        out_shape=[
            jax.ShapeDtypeStruct((NP, 144), jnp.float32),
            jax.ShapeDtypeStruct((NP, k), jnp.float32),
        ],
    )(s, R816, b.reshape(1, 128), W, A)


def _final_body(s_ref, b_ref, o_ref):
    acc = s_ref[0] + s_ref[1]
    msg = acc[:, 8:136]
    den = acc[:, 0:1]
    o_ref[...] = msg / (den + 1e-16) + b_ref[...]


def _final(s, b):
    return pl.pallas_call(
        _final_body,
        grid=(N // ROW_BLK,),
        in_specs=[
            pl.BlockSpec((2, ROW_BLK, ACCW), lambda i: (0, i, 0)),
            pl.BlockSpec((1, 128), lambda i: (0, 0)),
        ],
        out_specs=pl.BlockSpec((ROW_BLK, 128), lambda i: (i, 0)),
        out_shape=jax.ShapeDtypeStruct((N, 128), jnp.float32),
    )(s, b.reshape(1, 128))


def kernel(x, edge_index, W1, att_src1, att_dst1, b1, W2, att_src2, att_dst2, b2):
    n = x.shape[0]
    loop = jnp.arange(n, dtype=edge_index.dtype)
    pad = EP - (E + n)
    src_all = jnp.concatenate(
        [edge_index[0], loop, jnp.full((pad,), n, edge_index.dtype)])
    dst_all = jnp.concatenate(
        [edge_index[1], loop, jnp.full((pad,), n, edge_index.dtype)])

    H1, C1 = att_src1.shape
    eye1 = jnp.eye(H1, dtype=x.dtype)
    As1 = (eye1[:, None, :] * att_src1[:, :, None]).reshape(H1 * C1, H1)
    Ad1 = (eye1[:, None, :] * att_dst1[:, :, None]).reshape(H1 * C1, H1)
    A1 = jnp.concatenate([As1, Ad1], axis=1)
    A2 = jnp.concatenate(
        [att_src2.T, att_dst2.T,
         jnp.zeros((128, 14), x.dtype)], axis=1)
    R816 = jnp.repeat(jnp.eye(8, dtype=x.dtype), 16, axis=1)

    h, a1 = _mm_attn(x, W1, A1)
    s1 = _edge_sc_8(a1, h, src_all, dst_all)
    g, a2 = _norm_elu_mm(s1, R816, b1, W2, A2)
    s2 = _edge_sc_1(a2, g, src_all, dst_all)
    return _final(s2, b2)
